# baseline (speedup 1.0000x reference)
import functools

import jax
import jax.numpy as jnp
from jax import lax
from jax.experimental import pallas as pl
from jax.experimental.pallas import tpu as pltpu
from jax.experimental.pallas import tpu_sc as plsc

_B = 16384
_N = 512
_MAX_ELEM = 94
_NC = 2
_NS = 16
_NW = _NC * _NS
_ROWS_PER_W = _B // _NW
_CHUNK = 64
_NCHUNK = _ROWS_PER_W // _CHUNK
_GROUPS = _CHUNK // 16
_TBL = 256 * _MAX_ELEM


def _body(idx_hbm, tbl_hbm, out_hbm, idx_a, idx_b, tbl_v, out_v, sem_a, sem_b):
    wid = lax.axis_index("s") * _NC + lax.axis_index("c")
    row0 = wid * _ROWS_PER_W
    lane = lax.iota(jnp.int32, 16)
    zero = jnp.zeros((16,), jnp.float32)

    bufs = [idx_a, idx_b]
    sems = [sem_a, sem_b]

    def start(c, b):
        pltpu.async_copy(
            idx_hbm.at[pl.ds(row0 + c * _CHUNK, _CHUNK)],
            bufs[b], sems[b])

    def compute_chunk(c, buf):
        @plsc.parallel_loop(0, _GROUPS)
        def _group(g):

            def rowpair_body(rp, res):
                r0 = g * 16 + rp * 2
                r1 = r0 + 1

                @plsc.parallel_loop(0, _N // 64, unroll=4,
                                    carry=(zero, zero, zero, zero))
                def _accs(jj, accs):
                    a0, a1, b0, b1 = accs
                    base = jj * 64
                    v0 = buf[r0, pl.ds(base, 16)]
                    v1 = buf[r0, pl.ds(base + 16, 16)]
                    v2 = buf[r0, pl.ds(base + 32, 16)]
                    v3 = buf[r0, pl.ds(base + 48, 16)]
                    u0 = buf[r1, pl.ds(base, 16)]
                    u1 = buf[r1, pl.ds(base + 16, 16)]
                    u2 = buf[r1, pl.ds(base + 32, 16)]
                    u3 = buf[r1, pl.ds(base + 48, 16)]
                    a0 = a0 + plsc.load_gather(tbl_v, [v0 + (v1 << 8)])
                    a1 = a1 + plsc.load_gather(tbl_v, [v2 + (v3 << 8)])
                    b0 = b0 + plsc.load_gather(tbl_v, [u0 + (u1 << 8)])
                    b1 = b1 + plsc.load_gather(tbl_v, [u2 + (u3 << 8)])
                    return a0, a1, b0, b1

                a0, a1, b0, b1 = _accs
                inv_n = jnp.float32(1.0 / _N)
                tot0 = jnp.sum((a0 + a1) * inv_n)
                tot1 = jnp.sum((b0 + b1) * inv_n)
                res = jnp.where(lane == rp * 2, tot0, res)
                return jnp.where(lane == rp * 2 + 1, tot1, res)

            res = lax.fori_loop(0, 8, rowpair_body, zero)
            out_v[pl.ds(c * _CHUNK + g * 16, 16)] = res

    def wait(c, b):
        pltpu.make_async_copy(
            idx_hbm.at[pl.ds(row0 + c * _CHUNK, _CHUNK)],
            bufs[b], sems[b]).wait()

    start(0, 0)
    pltpu.sync_copy(tbl_hbm, tbl_v)

    @pl.loop(0, _NCHUNK, step=2)
    def _chunks(c):
        wait(c, 0)
        start(c + 1, 1)
        compute_chunk(c, bufs[0])
        wait(c + 1, 1)

        @pl.when(c + 2 < _NCHUNK)
        def _():
            start(c + 2, 0)

        compute_chunk(c + 1, bufs[1])

    pltpu.sync_copy(out_v, out_hbm.at[pl.ds(row0, _ROWS_PER_W)])


@jax.jit
def kernel(atomic_number, W):
    w = W.reshape(-1).astype(jnp.float32)
    wpad = jnp.zeros((256,), jnp.float32).at[:_MAX_ELEM].set(w)
    tbl = (w[:, None] + wpad[None, :]).reshape(-1)

    mesh = plsc.VectorSubcoreMesh(core_axis_name="c", subcore_axis_name="s")
    run = functools.partial(
        pl.kernel,
        mesh=mesh,
        out_type=jax.ShapeDtypeStruct((_B,), jnp.float32),
        scratch_types=[
            pltpu.VMEM((_CHUNK, _N), jnp.int32),
            pltpu.VMEM((_CHUNK, _N), jnp.int32),
            pltpu.VMEM((_TBL,), jnp.float32),
            pltpu.VMEM((_ROWS_PER_W,), jnp.float32),
            pltpu.SemaphoreType.DMA,
            pltpu.SemaphoreType.DMA,
        ],
        compiler_params=pltpu.CompilerParams(
            needs_layout_passes=False, use_tc_tiling_on_sc=True),
    )(_body)
    return run(atomic_number, tbl)


# speedup vs baseline: 1.1142x; 1.1142x over previous
import functools

import jax
import jax.numpy as jnp
from jax import lax
from jax.experimental import pallas as pl
from jax.experimental.pallas import tpu as pltpu
from jax.experimental.pallas import tpu_sc as plsc

_B = 16384
_N = 512
_MAX_ELEM = 94
_NC = 2
_NS = 16
_NW = _NC * _NS
_ROWS_PER_W = _B // _NW
_CHUNK = 64
_NCHUNK = _ROWS_PER_W // _CHUNK
_GROUPS = _CHUNK // 16
_TBL = 256 * _MAX_ELEM


def _body(idx_hbm, tbl_hbm, out_hbm, idx_a, idx_b, tbl_v, out_v, sem_a, sem_b):
    wid = lax.axis_index("s") * _NC + lax.axis_index("c")
    row0 = wid * _ROWS_PER_W
    lane = lax.iota(jnp.int32, 16)
    zero = jnp.zeros((16,), jnp.float32)

    bufs = [idx_a, idx_b]
    sems = [sem_a, sem_b]

    def start(c, b):
        pltpu.async_copy(
            idx_hbm.at[pl.ds(row0 + c * _CHUNK, _CHUNK)],
            bufs[b], sems[b])

    def compute_chunk(c, buf):
        @plsc.parallel_loop(0, _GROUPS)
        def _group(g):

            def rowpair_body(rp, res):
                r0 = g * 16 + rp * 2
                r1 = r0 + 1

                @plsc.parallel_loop(0, _N // 64, unroll=8,
                                    carry=(zero, zero, zero, zero))
                def _accs(jj, accs):
                    a0, a1, b0, b1 = accs
                    base = jj * 64
                    v0 = buf[r0, pl.ds(base, 16)]
                    v1 = buf[r0, pl.ds(base + 16, 16)]
                    v2 = buf[r0, pl.ds(base + 32, 16)]
                    v3 = buf[r0, pl.ds(base + 48, 16)]
                    u0 = buf[r1, pl.ds(base, 16)]
                    u1 = buf[r1, pl.ds(base + 16, 16)]
                    u2 = buf[r1, pl.ds(base + 32, 16)]
                    u3 = buf[r1, pl.ds(base + 48, 16)]
                    a0 = a0 + plsc.load_gather(tbl_v, [v0 + (v1 << 8)])
                    a1 = a1 + plsc.load_gather(tbl_v, [v2 + (v3 << 8)])
                    b0 = b0 + plsc.load_gather(tbl_v, [u0 + (u1 << 8)])
                    b1 = b1 + plsc.load_gather(tbl_v, [u2 + (u3 << 8)])
                    return a0, a1, b0, b1

                a0, a1, b0, b1 = _accs
                inv_n = jnp.float32(1.0 / _N)
                tot0 = jnp.sum((a0 + a1) * inv_n)
                tot1 = jnp.sum((b0 + b1) * inv_n)
                res = jnp.where(lane == rp * 2, tot0, res)
                return jnp.where(lane == rp * 2 + 1, tot1, res)

            res = lax.fori_loop(0, 8, rowpair_body, zero)
            out_v[pl.ds(c * _CHUNK + g * 16, 16)] = res

    def wait(c, b):
        pltpu.make_async_copy(
            idx_hbm.at[pl.ds(row0 + c * _CHUNK, _CHUNK)],
            bufs[b], sems[b]).wait()

    start(0, 0)
    pltpu.sync_copy(tbl_hbm, tbl_v)

    @pl.loop(0, _NCHUNK, step=2)
    def _chunks(c):
        wait(c, 0)
        start(c + 1, 1)
        compute_chunk(c, bufs[0])
        wait(c + 1, 1)

        @pl.when(c + 2 < _NCHUNK)
        def _():
            start(c + 2, 0)

        compute_chunk(c + 1, bufs[1])

    pltpu.sync_copy(out_v, out_hbm.at[pl.ds(row0, _ROWS_PER_W)])


@jax.jit
def kernel(atomic_number, W):
    w = W.reshape(-1).astype(jnp.float32)
    wpad = jnp.zeros((256,), jnp.float32).at[:_MAX_ELEM].set(w)
    tbl = (w[:, None] + wpad[None, :]).reshape(-1)

    mesh = plsc.VectorSubcoreMesh(core_axis_name="c", subcore_axis_name="s")
    run = functools.partial(
        pl.kernel,
        mesh=mesh,
        out_type=jax.ShapeDtypeStruct((_B,), jnp.float32),
        scratch_types=[
            pltpu.VMEM((_CHUNK, _N), jnp.int32),
            pltpu.VMEM((_CHUNK, _N), jnp.int32),
            pltpu.VMEM((_TBL,), jnp.float32),
            pltpu.VMEM((_ROWS_PER_W,), jnp.float32),
            pltpu.SemaphoreType.DMA,
            pltpu.SemaphoreType.DMA,
        ],
        compiler_params=pltpu.CompilerParams(
            needs_layout_passes=False, use_tc_tiling_on_sc=True),
    )(_body)
    return run(atomic_number, tbl)


# speedup vs baseline: 1.1175x; 1.0030x over previous
import functools

import jax
import jax.numpy as jnp
from jax import lax
from jax.experimental import pallas as pl
from jax.experimental.pallas import tpu as pltpu
from jax.experimental.pallas import tpu_sc as plsc

_B = 16384
_N = 512
_MAX_ELEM = 94
_NC = 2
_NS = 16
_NW = _NC * _NS
_ROWS_PER_W = _B // _NW
_CHUNK = 64
_NCHUNK = _ROWS_PER_W // _CHUNK
_GROUPS = _CHUNK // 16
_TBL = 256 * _MAX_ELEM


def _body(idx_hbm, tbl_hbm, out_hbm, idx_a, idx_b, tbl_v, out_v, sem_a, sem_b):
    wid = lax.axis_index("s") * _NC + lax.axis_index("c")
    row0 = wid * _ROWS_PER_W
    lane = lax.iota(jnp.int32, 16)
    zero = jnp.zeros((16,), jnp.float32)

    bufs = [idx_a, idx_b]
    sems = [sem_a, sem_b]

    def start(c, b):
        pltpu.async_copy(
            idx_hbm.at[pl.ds(row0 + c * _CHUNK, _CHUNK)],
            bufs[b], sems[b])

    def compute_chunk(c, buf):
        @plsc.parallel_loop(0, _GROUPS)
        def _group(g):

            @plsc.parallel_loop(0, 8, carry=zero)
            def res(rp, res):
                r0 = g * 16 + rp * 2
                r1 = r0 + 1

                @plsc.parallel_loop(0, _N // 64, unroll=8,
                                    carry=(zero, zero, zero, zero))
                def _accs(jj, accs):
                    a0, a1, b0, b1 = accs
                    base = jj * 64
                    v0 = buf[r0, pl.ds(base, 16)]
                    v1 = buf[r0, pl.ds(base + 16, 16)]
                    v2 = buf[r0, pl.ds(base + 32, 16)]
                    v3 = buf[r0, pl.ds(base + 48, 16)]
                    u0 = buf[r1, pl.ds(base, 16)]
                    u1 = buf[r1, pl.ds(base + 16, 16)]
                    u2 = buf[r1, pl.ds(base + 32, 16)]
                    u3 = buf[r1, pl.ds(base + 48, 16)]
                    a0 = a0 + plsc.load_gather(tbl_v, [v0 + (v1 << 8)])
                    a1 = a1 + plsc.load_gather(tbl_v, [v2 + (v3 << 8)])
                    b0 = b0 + plsc.load_gather(tbl_v, [u0 + (u1 << 8)])
                    b1 = b1 + plsc.load_gather(tbl_v, [u2 + (u3 << 8)])
                    return a0, a1, b0, b1

                a0, a1, b0, b1 = _accs
                inv_n = jnp.float32(1.0 / _N)
                tot0 = jnp.sum((a0 + a1) * inv_n)
                tot1 = jnp.sum((b0 + b1) * inv_n)
                res = jnp.where(lane == rp * 2, tot0, res)
                return jnp.where(lane == rp * 2 + 1, tot1, res)

            out_v[pl.ds(c * _CHUNK + g * 16, 16)] = res

    def wait(c, b):
        pltpu.make_async_copy(
            idx_hbm.at[pl.ds(row0 + c * _CHUNK, _CHUNK)],
            bufs[b], sems[b]).wait()

    start(0, 0)
    pltpu.sync_copy(tbl_hbm, tbl_v)

    @pl.loop(0, _NCHUNK, step=2)
    def _chunks(c):
        wait(c, 0)
        start(c + 1, 1)
        compute_chunk(c, bufs[0])
        wait(c + 1, 1)

        @pl.when(c + 2 < _NCHUNK)
        def _():
            start(c + 2, 0)

        compute_chunk(c + 1, bufs[1])

    pltpu.sync_copy(out_v, out_hbm.at[pl.ds(row0, _ROWS_PER_W)])


@jax.jit
def kernel(atomic_number, W):
    w = W.reshape(-1).astype(jnp.float32)
    wpad = jnp.zeros((256,), jnp.float32).at[:_MAX_ELEM].set(w)
    tbl = (w[:, None] + wpad[None, :]).reshape(-1)

    mesh = plsc.VectorSubcoreMesh(core_axis_name="c", subcore_axis_name="s")
    run = functools.partial(
        pl.kernel,
        mesh=mesh,
        out_type=jax.ShapeDtypeStruct((_B,), jnp.float32),
        scratch_types=[
            pltpu.VMEM((_CHUNK, _N), jnp.int32),
            pltpu.VMEM((_CHUNK, _N), jnp.int32),
            pltpu.VMEM((_TBL,), jnp.float32),
            pltpu.VMEM((_ROWS_PER_W,), jnp.float32),
            pltpu.SemaphoreType.DMA,
            pltpu.SemaphoreType.DMA,
        ],
        compiler_params=pltpu.CompilerParams(
            needs_layout_passes=False, use_tc_tiling_on_sc=True),
    )(_body)
    return run(atomic_number, tbl)
